# baseline (speedup 1.0000x reference)
import functools

import jax
import jax.numpy as jnp
from jax import lax
from jax.experimental import pallas as pl
from jax.experimental.pallas import tpu as pltpu
from jax.experimental.pallas import tpu_sc as plsc

N_ = 10000
E_ = 160000
D_ = 256
L_ = 3
B_ = 128

NPAD = 10240
BLK = 512
NB = NPAD // BLK
H = 128
CH = 128
NCHUNK = 80
EPT = NCHUNK * CH
EPAD = 16 * EPT
TROW = NPAD - 1
RPT = NPAD // 16
NVALF = float(N_)


def _sc_body(h_hbm, src_hbm, dst_hbm, zeros_hbm, out_hbm,
             sidx0, sidx1, didx0, didx1, rows0, rows1, agg_sh, sem0, sem1):
    c = lax.axis_index("c")
    s = lax.axis_index("s")
    r0 = s * RPT
    pltpu.sync_copy(zeros_hbm.at[pl.ds(r0, RPT)], agg_sh.at[pl.ds(r0, RPT)])

    coff = c * NPAD
    ebase = s * EPT

    def load_idx(k, sidx, didx):
        pltpu.sync_copy(src_hbm.at[pl.ds(ebase + k * CH, CH)], sidx)
        for i in range(CH // 16):
            sl = pl.ds(i * 16, 16)
            sidx[sl] = sidx[sl] + coff
        pltpu.sync_copy(dst_hbm.at[pl.ds(ebase + k * CH, CH)], didx.at[0])

    plsc.subcore_barrier()

    load_idx(0, sidx0, didx0)
    pltpu.async_copy(h_hbm.at[sidx0], rows0, sem0)

    def pair(kk, carry):
        k0 = 2 * kk
        load_idx(k0 + 1, sidx1, didx1)
        pltpu.async_copy(h_hbm.at[sidx1], rows1, sem1)
        pltpu.make_async_copy(h_hbm.at[sidx0], rows0, sem0).wait()
        pltpu.sync_copy(rows0, agg_sh.at[didx0.at[0]], add=True)
        knext = jnp.minimum(k0 + 2, NCHUNK - 1)
        load_idx(knext, sidx0, didx0)
        pltpu.async_copy(h_hbm.at[sidx0], rows0, sem0)
        pltpu.make_async_copy(h_hbm.at[sidx1], rows1, sem1).wait()
        pltpu.sync_copy(rows1, agg_sh.at[didx1.at[0]], add=True)
        return carry

    lax.fori_loop(0, NCHUNK // 2, pair, 0)
    pltpu.make_async_copy(h_hbm.at[sidx0], rows0, sem0).wait()
    plsc.subcore_barrier()
    pltpu.sync_copy(agg_sh.at[pl.ds(r0, RPT)],
                    out_hbm.at[pl.ds(coff + r0, RPT)])


@functools.cache
def _build_segsum():
    return pl.kernel(
        _sc_body,
        mesh=plsc.VectorSubcoreMesh(core_axis_name="c", subcore_axis_name="s"),
        out_type=jax.ShapeDtypeStruct((2 * NPAD, H), jnp.float32),
        scratch_types=[
            pltpu.VMEM((CH,), jnp.int32),
            pltpu.VMEM((CH,), jnp.int32),
            pltpu.VMEM((1, CH), jnp.int32),
            pltpu.VMEM((1, CH), jnp.int32),
            pltpu.VMEM((CH, H), jnp.float32),
            pltpu.VMEM((CH, H), jnp.float32),
            pltpu.VMEM_SHARED((NPAD, H), jnp.float32),
            pltpu.SemaphoreType.DMA,
            pltpu.SemaphoreType.DMA,
        ],
    )


def _segsum(h2, srcp, dstp, zeros128):
    return _build_segsum()(h2, srcp, dstp, zeros128)


def _acc_stats(i, val8, ref):
    @pl.when(i == 0)
    def _():
        ref[...] = val8

    @pl.when(i != 0)
    def _():
        ref[...] += val8


def _masked_sums(i, t):
    rid = lax.broadcasted_iota(jnp.int32, (BLK, 1), 0) + i * BLK
    tm = jnp.where(rid < N_, t, 0.0)
    ps = jnp.broadcast_to(jnp.sum(tm, axis=0, keepdims=True), (8, t.shape[1]))
    pq = jnp.broadcast_to(jnp.sum(tm * tm, axis=0, keepdims=True),
                          (8, t.shape[1]))
    return ps, pq


def _p0_body(x_ref, w_ref, b_ref, o_ref):
    o_ref[...] = (jnp.dot(x_ref[...], w_ref[...],
                          preferred_element_type=jnp.float32) + b_ref[...])


_p0 = pl.pallas_call(
    _p0_body,
    grid=(2, NB),
    in_specs=[
        pl.BlockSpec((BLK, D_), lambda h, i: (i, 0)),
        pl.BlockSpec((D_, H), lambda h, i: (0, h)),
        pl.BlockSpec((1, H), lambda h, i: (0, h)),
    ],
    out_specs=pl.BlockSpec((BLK, H), lambda h, i: (h * NB + i, 0)),
    out_shape=jax.ShapeDtypeStruct((2 * NPAD, H), jnp.float32),
)


def _p1_body(eps_ref, ha_ref, hb_ref, aa_ref, ab_ref, w_ref, b_ref,
             t_ref, s_ref, q_ref):
    i = pl.program_id(0)
    e = eps_ref[0, 0]
    za = ha_ref[...] * e + aa_ref[...]
    zb = hb_ref[...] * e + ab_ref[...]
    w = w_ref[...]
    t = (jnp.dot(za, w[:H, :], preferred_element_type=jnp.float32)
         + jnp.dot(zb, w[H:, :], preferred_element_type=jnp.float32)
         + b_ref[...])
    t_ref[...] = t
    ps, pq = _masked_sums(i, t)
    _acc_stats(i, ps, s_ref)
    _acc_stats(i, pq, q_ref)


_p1 = pl.pallas_call(
    _p1_body,
    grid=(NB,),
    in_specs=[
        pl.BlockSpec(memory_space=pltpu.SMEM),
        pl.BlockSpec((BLK, H), lambda i: (i, 0)),
        pl.BlockSpec((BLK, H), lambda i: (NB + i, 0)),
        pl.BlockSpec((BLK, H), lambda i: (i, 0)),
        pl.BlockSpec((BLK, H), lambda i: (NB + i, 0)),
        pl.BlockSpec((D_, 2 * D_), lambda i: (0, 0)),
        pl.BlockSpec((1, 2 * D_), lambda i: (0, 0)),
    ],
    out_specs=[
        pl.BlockSpec((BLK, 2 * D_), lambda i: (i, 0)),
        pl.BlockSpec((8, 2 * D_), lambda i: (0, 0)),
        pl.BlockSpec((8, 2 * D_), lambda i: (0, 0)),
    ],
    out_shape=[
        jax.ShapeDtypeStruct((NPAD, 2 * D_), jnp.float32),
        jax.ShapeDtypeStruct((8, 2 * D_), jnp.float32),
        jax.ShapeDtypeStruct((8, 2 * D_), jnp.float32),
    ],
)


def _p2_body(t_ref, s_ref, q_ref, g_ref, be_ref, w_ref, b_ref,
             u_ref, us_ref, uq_ref):
    i = pl.program_id(0)
    mu = s_ref[0:1, :] * (1.0 / NVALF)
    var = q_ref[0:1, :] * (1.0 / NVALF) - mu * mu
    inv = lax.rsqrt(var + 1e-5)
    tn = (t_ref[...] - mu) * (inv * g_ref[...]) + be_ref[...]
    r = jnp.maximum(tn, 0.0)
    u = (jnp.dot(r, w_ref[...], preferred_element_type=jnp.float32)
         + b_ref[...])
    u_ref[...] = u
    ps, pq = _masked_sums(i, u)
    _acc_stats(i, ps, us_ref)
    _acc_stats(i, pq, uq_ref)


_p2 = pl.pallas_call(
    _p2_body,
    grid=(NB,),
    in_specs=[
        pl.BlockSpec((BLK, 2 * D_), lambda i: (i, 0)),
        pl.BlockSpec((8, 2 * D_), lambda i: (0, 0)),
        pl.BlockSpec((8, 2 * D_), lambda i: (0, 0)),
        pl.BlockSpec((1, 2 * D_), lambda i: (0, 0)),
        pl.BlockSpec((1, 2 * D_), lambda i: (0, 0)),
        pl.BlockSpec((2 * D_, D_), lambda i: (0, 0)),
        pl.BlockSpec((1, D_), lambda i: (0, 0)),
    ],
    out_specs=[
        pl.BlockSpec((BLK, D_), lambda i: (i, 0)),
        pl.BlockSpec((8, D_), lambda i: (0, 0)),
        pl.BlockSpec((8, D_), lambda i: (0, 0)),
    ],
    out_shape=[
        jax.ShapeDtypeStruct((NPAD, D_), jnp.float32),
        jax.ShapeDtypeStruct((8, D_), jnp.float32),
        jax.ShapeDtypeStruct((8, D_), jnp.float32),
    ],
)


def _p3_body(u_ref, s_ref, q_ref, g_ref, be_ref, o_ref):
    mu = s_ref[0:1, :] * (1.0 / NVALF)
    var = q_ref[0:1, :] * (1.0 / NVALF) - mu * mu
    inv = lax.rsqrt(var + 1e-5)
    o_ref[...] = jnp.maximum(
        (u_ref[...] - mu) * (inv * g_ref[...]) + be_ref[...], 0.0)


_p3 = pl.pallas_call(
    _p3_body,
    grid=(2, NB),
    in_specs=[
        pl.BlockSpec((BLK, H), lambda h, i: (i, h)),
        pl.BlockSpec((8, H), lambda h, i: (0, h)),
        pl.BlockSpec((8, H), lambda h, i: (0, h)),
        pl.BlockSpec((1, H), lambda h, i: (0, h)),
        pl.BlockSpec((1, H), lambda h, i: (0, h)),
    ],
    out_specs=pl.BlockSpec((BLK, H), lambda h, i: (h * NB + i, 0)),
    out_shape=jax.ShapeDtypeStruct((2 * NPAD, H), jnp.float32),
)


def _ps1_body(ha_ref, hb_ref, w_ref, b_ref, t_ref, s_ref, q_ref):
    i = pl.program_id(0)
    w = w_ref[...]
    t = (jnp.dot(ha_ref[...], w[:H, :], preferred_element_type=jnp.float32)
         + jnp.dot(hb_ref[...], w[H:, :], preferred_element_type=jnp.float32)
         + b_ref[...])
    t_ref[...] = t
    ps, pq = _masked_sums(i, t)
    _acc_stats(i, ps, s_ref)
    _acc_stats(i, pq, q_ref)


_ps1 = pl.pallas_call(
    _ps1_body,
    grid=(NB,),
    in_specs=[
        pl.BlockSpec((BLK, H), lambda i: (i, 0)),
        pl.BlockSpec((BLK, H), lambda i: (NB + i, 0)),
        pl.BlockSpec((D_, 2 * D_), lambda i: (0, 0)),
        pl.BlockSpec((1, 2 * D_), lambda i: (0, 0)),
    ],
    out_specs=[
        pl.BlockSpec((BLK, 2 * D_), lambda i: (i, 0)),
        pl.BlockSpec((8, 2 * D_), lambda i: (0, 0)),
        pl.BlockSpec((8, 2 * D_), lambda i: (0, 0)),
    ],
    out_shape=[
        jax.ShapeDtypeStruct((NPAD, 2 * D_), jnp.float32),
        jax.ShapeDtypeStruct((8, 2 * D_), jnp.float32),
        jax.ShapeDtypeStruct((8, 2 * D_), jnp.float32),
    ],
)


def _ps2_body(s_ref, ss_ref, sq_ref, g_ref, be_ref, w_ref, b_ref, bt_ref,
              sc_ref, pb_ref, nb_ref):
    i = pl.program_id(0)
    mu = ss_ref[0:1, :] * (1.0 / NVALF)
    var = sq_ref[0:1, :] * (1.0 / NVALF) - mu * mu
    inv = lax.rsqrt(var + 1e-5)
    sn = (s_ref[...] - mu) * (inv * g_ref[...]) + be_ref[...]
    r = jnp.maximum(sn, 0.0)
    v = (jnp.dot(r, w_ref[...], preferred_element_type=jnp.float32)
         + b_ref[...])
    score = jax.nn.sigmoid(v)
    sc_ref[...] = score
    pos = jnp.mean(score, axis=1, keepdims=True)
    bcol = bt_ref[...]
    oneh = (bcol == lax.broadcasted_iota(jnp.int32, (1, B_), 1))
    oneh = oneh.astype(jnp.float32)
    pp = jnp.broadcast_to(
        jnp.sum(pos * oneh, axis=0, keepdims=True), (8, B_))
    nn = jnp.broadcast_to(
        jnp.sum((1.0 - pos) * oneh, axis=0, keepdims=True), (8, B_))

    @pl.when(i == 0)
    def _():
        pb_ref[...] = pp + 1e-8
        nb_ref[...] = nn + 1e-8

    @pl.when(i != 0)
    def _():
        pb_ref[...] += pp
        nb_ref[...] += nn


_ps2 = pl.pallas_call(
    _ps2_body,
    grid=(NB,),
    in_specs=[
        pl.BlockSpec((BLK, 2 * D_), lambda i: (i, 0)),
        pl.BlockSpec((8, 2 * D_), lambda i: (0, 0)),
        pl.BlockSpec((8, 2 * D_), lambda i: (0, 0)),
        pl.BlockSpec((1, 2 * D_), lambda i: (0, 0)),
        pl.BlockSpec((1, 2 * D_), lambda i: (0, 0)),
        pl.BlockSpec((2 * D_, D_), lambda i: (0, 0)),
        pl.BlockSpec((1, D_), lambda i: (0, 0)),
        pl.BlockSpec((BLK, 1), lambda i: (i, 0)),
    ],
    out_specs=[
        pl.BlockSpec((BLK, D_), lambda i: (i, 0)),
        pl.BlockSpec((8, B_), lambda i: (0, 0)),
        pl.BlockSpec((8, B_), lambda i: (0, 0)),
    ],
    out_shape=[
        jax.ShapeDtypeStruct((NPAD, D_), jnp.float32),
        jax.ShapeDtypeStruct((8, B_), jnp.float32),
        jax.ShapeDtypeStruct((8, B_), jnp.float32),
    ],
)


def kernel(x, edge_index, batch, W0, b0, W1, b1, g1, be1, W2, b2, g2, be2,
           eps, Ws1, bs1, gs1, bes1, Ws2, bs2):
    f32 = jnp.float32
    xp = jnp.pad(x, ((0, NPAD - N_), (0, 0)))
    src = edge_index[0].astype(jnp.int32)
    dst = edge_index[1].astype(jnp.int32)
    srcp = jnp.concatenate([src, jnp.zeros((EPAD - E_,), jnp.int32)])
    dstp = jnp.concatenate([dst, jnp.full((EPAD - E_,), TROW, jnp.int32)])
    batchp = jnp.pad(batch.astype(jnp.int32), (0, NPAD - N_),
                     constant_values=B_)[:, None]
    zeros128 = jnp.zeros((NPAD, H), f32)

    h2 = _p0(xp, W0, b0.reshape(1, D_))
    for l in range(L_):
        agg2 = _segsum(h2, srcp, dstp, zeros128)
        epsl = (1.0 + eps[l]).astype(f32).reshape(1, 1)
        t, ts, tq = _p1(epsl, h2, h2, agg2, agg2, W1[l], b1[l].reshape(1, -1))
        u, us, uq = _p2(t, ts, tq, g1[l].reshape(1, -1), be1[l].reshape(1, -1),
                        W2[l], b2[l].reshape(1, -1))
        h2 = _p3(u, us, uq, g2[l].reshape(1, -1), be2[l].reshape(1, -1))

    s, ss, sq = _ps1(h2, h2, Ws1, bs1.reshape(1, -1))
    scorep, pb, nb = _ps2(s, ss, sq, gs1.reshape(1, -1), bes1.reshape(1, -1),
                          Ws2, bs2.reshape(1, -1), batchp)
    return scorep[:N_], pb[0], nb[0]


# speedup vs baseline: 1.0356x; 1.0356x over previous
import functools

import jax
import jax.numpy as jnp
from jax import lax
from jax.experimental import pallas as pl
from jax.experimental.pallas import tpu as pltpu
from jax.experimental.pallas import tpu_sc as plsc

N_ = 10000
E_ = 160000
D_ = 256
L_ = 3
B_ = 128

NPAD = 10240
BLK = 512
NB = NPAD // BLK
H = 128
CH = 64
NCHUNK = 160
EPT = NCHUNK * CH
EPAD = 16 * EPT
TROW = NPAD - 1
RPT = NPAD // 16
NVALF = float(N_)


NBUF = 4
PROW = NCHUNK // 2


def _sc_body(h_hbm, pk_hbm, zeros_hbm, out_hbm,
             pk, si0, si1, si2, si3, di0, di1, di2, di3,
             r0b, r1b, r2b, r3b, agg_sh,
             sg0, sg1, sg2, sg3, ss0, ss1, ss2, ss3):
    c = lax.axis_index("c")
    s = lax.axis_index("s")
    row0 = s * RPT
    coff = c * NPAD
    pltpu.sync_copy(zeros_hbm.at[pl.ds(row0, RPT)],
                    agg_sh.at[pl.ds(row0, RPT)])
    pltpu.sync_copy(pk_hbm.at[s], pk)
    plsc.subcore_barrier()

    rows = [r0b, r1b, r2b, r3b]
    sis = [si0, si1, si2, si3]
    dis = [di0, di1, di2, di3]
    sg = [sg0, sg1, sg2, sg3]
    ss = [ss0, ss1, ss2, ss3]
    gd = [None] * NBUF
    sd = [None] * NBUF

    def unpack(k):
        b = k % NBUF
        q, half = k // 2, (k % 2) * CH
        for i in range(CH // 16):
            sl = pl.ds(half + i * 16, 16)
            so = pl.ds(i * 16, 16)
            p = pk[q, sl]
            sis[b][0, so] = (p & 0xFFFF) + coff
            dis[b][0, so] = p >> 16

    def start_g(k):
        b = k % NBUF
        gd[b] = pltpu.async_copy(h_hbm.at[sis[b].at[0]], rows[b], sg[b])

    def start_s(k):
        b = k % NBUF
        sd[b] = pltpu.async_copy(rows[b], agg_sh.at[dis[b].at[0]], ss[b],
                                 add=True)

    unpack(0)
    start_g(0)
    unpack(1)
    start_g(1)
    for k in range(NCHUNK):
        b = k % NBUF
        gd[b].wait()
        start_s(k)
        kn = k + 2
        if kn < NCHUNK:
            bn = kn % NBUF
            if sd[bn] is not None:
                sd[bn].wait()
                sd[bn] = None
            unpack(kn)
            start_g(kn)
    for b in range(NBUF):
        if sd[b] is not None:
            sd[b].wait()
    plsc.subcore_barrier()
    pltpu.sync_copy(agg_sh.at[pl.ds(row0, RPT)],
                    out_hbm.at[pl.ds(c * NPAD + row0, RPT)])


@functools.cache
def _build_segsum():
    return pl.kernel(
        _sc_body,
        mesh=plsc.VectorSubcoreMesh(core_axis_name="c", subcore_axis_name="s"),
        out_type=jax.ShapeDtypeStruct((2 * NPAD, H), jnp.float32),
        scratch_types=(
            [pltpu.VMEM((PROW, 2 * CH), jnp.int32)]
            + [pltpu.VMEM((1, CH), jnp.int32) for _ in range(2 * NBUF)]
            + [pltpu.VMEM((CH, H), jnp.float32) for _ in range(NBUF)]
            + [pltpu.VMEM_SHARED((NPAD, H), jnp.float32)]
            + [pltpu.SemaphoreType.DMA for _ in range(2 * NBUF)]
        ),
    )


def _segsum(h2, pk, zeros128):
    return _build_segsum()(h2, pk, zeros128)


def _acc_stats(i, val8, ref):
    @pl.when(i == 0)
    def _():
        ref[...] = val8

    @pl.when(i != 0)
    def _():
        ref[...] += val8


def _masked_sums(i, t):
    rid = lax.broadcasted_iota(jnp.int32, (BLK, 1), 0) + i * BLK
    tm = jnp.where(rid < N_, t, 0.0)
    ps = jnp.broadcast_to(jnp.sum(tm, axis=0, keepdims=True), (8, t.shape[1]))
    pq = jnp.broadcast_to(jnp.sum(tm * tm, axis=0, keepdims=True),
                          (8, t.shape[1]))
    return ps, pq


def _p0_body(x_ref, w_ref, b_ref, o_ref):
    o_ref[...] = (jnp.dot(x_ref[...], w_ref[...],
                          preferred_element_type=jnp.float32) + b_ref[...])


_p0 = pl.pallas_call(
    _p0_body,
    grid=(2, NB),
    in_specs=[
        pl.BlockSpec((BLK, D_), lambda h, i: (i, 0)),
        pl.BlockSpec((D_, H), lambda h, i: (0, h)),
        pl.BlockSpec((1, H), lambda h, i: (0, h)),
    ],
    out_specs=pl.BlockSpec((BLK, H), lambda h, i: (h * NB + i, 0)),
    out_shape=jax.ShapeDtypeStruct((2 * NPAD, H), jnp.float32),
)


def _p1_body(eps_ref, ha_ref, hb_ref, aa_ref, ab_ref, w_ref, b_ref,
             t_ref, s_ref, q_ref):
    i = pl.program_id(0)
    e = eps_ref[0, 0]
    za = ha_ref[...] * e + aa_ref[...]
    zb = hb_ref[...] * e + ab_ref[...]
    w = w_ref[...]
    t = (jnp.dot(za, w[:H, :], preferred_element_type=jnp.float32)
         + jnp.dot(zb, w[H:, :], preferred_element_type=jnp.float32)
         + b_ref[...])
    t_ref[...] = t
    ps, pq = _masked_sums(i, t)
    _acc_stats(i, ps, s_ref)
    _acc_stats(i, pq, q_ref)


_p1 = pl.pallas_call(
    _p1_body,
    grid=(NB,),
    in_specs=[
        pl.BlockSpec(memory_space=pltpu.SMEM),
        pl.BlockSpec((BLK, H), lambda i: (i, 0)),
        pl.BlockSpec((BLK, H), lambda i: (NB + i, 0)),
        pl.BlockSpec((BLK, H), lambda i: (i, 0)),
        pl.BlockSpec((BLK, H), lambda i: (NB + i, 0)),
        pl.BlockSpec((D_, 2 * D_), lambda i: (0, 0)),
        pl.BlockSpec((1, 2 * D_), lambda i: (0, 0)),
    ],
    out_specs=[
        pl.BlockSpec((BLK, 2 * D_), lambda i: (i, 0)),
        pl.BlockSpec((8, 2 * D_), lambda i: (0, 0)),
        pl.BlockSpec((8, 2 * D_), lambda i: (0, 0)),
    ],
    out_shape=[
        jax.ShapeDtypeStruct((NPAD, 2 * D_), jnp.float32),
        jax.ShapeDtypeStruct((8, 2 * D_), jnp.float32),
        jax.ShapeDtypeStruct((8, 2 * D_), jnp.float32),
    ],
)


def _p2_body(t_ref, s_ref, q_ref, g_ref, be_ref, w_ref, b_ref,
             u_ref, us_ref, uq_ref):
    i = pl.program_id(0)
    mu = s_ref[0:1, :] * (1.0 / NVALF)
    var = q_ref[0:1, :] * (1.0 / NVALF) - mu * mu
    inv = lax.rsqrt(var + 1e-5)
    tn = (t_ref[...] - mu) * (inv * g_ref[...]) + be_ref[...]
    r = jnp.maximum(tn, 0.0)
    u = (jnp.dot(r, w_ref[...], preferred_element_type=jnp.float32)
         + b_ref[...])
    u_ref[...] = u
    ps, pq = _masked_sums(i, u)
    _acc_stats(i, ps, us_ref)
    _acc_stats(i, pq, uq_ref)


_p2 = pl.pallas_call(
    _p2_body,
    grid=(NB,),
    in_specs=[
        pl.BlockSpec((BLK, 2 * D_), lambda i: (i, 0)),
        pl.BlockSpec((8, 2 * D_), lambda i: (0, 0)),
        pl.BlockSpec((8, 2 * D_), lambda i: (0, 0)),
        pl.BlockSpec((1, 2 * D_), lambda i: (0, 0)),
        pl.BlockSpec((1, 2 * D_), lambda i: (0, 0)),
        pl.BlockSpec((2 * D_, D_), lambda i: (0, 0)),
        pl.BlockSpec((1, D_), lambda i: (0, 0)),
    ],
    out_specs=[
        pl.BlockSpec((BLK, D_), lambda i: (i, 0)),
        pl.BlockSpec((8, D_), lambda i: (0, 0)),
        pl.BlockSpec((8, D_), lambda i: (0, 0)),
    ],
    out_shape=[
        jax.ShapeDtypeStruct((NPAD, D_), jnp.float32),
        jax.ShapeDtypeStruct((8, D_), jnp.float32),
        jax.ShapeDtypeStruct((8, D_), jnp.float32),
    ],
)


def _p3_body(u_ref, s_ref, q_ref, g_ref, be_ref, o_ref):
    mu = s_ref[0:1, :] * (1.0 / NVALF)
    var = q_ref[0:1, :] * (1.0 / NVALF) - mu * mu
    inv = lax.rsqrt(var + 1e-5)
    o_ref[...] = jnp.maximum(
        (u_ref[...] - mu) * (inv * g_ref[...]) + be_ref[...], 0.0)


_p3 = pl.pallas_call(
    _p3_body,
    grid=(2, NB),
    in_specs=[
        pl.BlockSpec((BLK, H), lambda h, i: (i, h)),
        pl.BlockSpec((8, H), lambda h, i: (0, h)),
        pl.BlockSpec((8, H), lambda h, i: (0, h)),
        pl.BlockSpec((1, H), lambda h, i: (0, h)),
        pl.BlockSpec((1, H), lambda h, i: (0, h)),
    ],
    out_specs=pl.BlockSpec((BLK, H), lambda h, i: (h * NB + i, 0)),
    out_shape=jax.ShapeDtypeStruct((2 * NPAD, H), jnp.float32),
)


def _ps1_body(ha_ref, hb_ref, w_ref, b_ref, t_ref, s_ref, q_ref):
    i = pl.program_id(0)
    w = w_ref[...]
    t = (jnp.dot(ha_ref[...], w[:H, :], preferred_element_type=jnp.float32)
         + jnp.dot(hb_ref[...], w[H:, :], preferred_element_type=jnp.float32)
         + b_ref[...])
    t_ref[...] = t
    ps, pq = _masked_sums(i, t)
    _acc_stats(i, ps, s_ref)
    _acc_stats(i, pq, q_ref)


_ps1 = pl.pallas_call(
    _ps1_body,
    grid=(NB,),
    in_specs=[
        pl.BlockSpec((BLK, H), lambda i: (i, 0)),
        pl.BlockSpec((BLK, H), lambda i: (NB + i, 0)),
        pl.BlockSpec((D_, 2 * D_), lambda i: (0, 0)),
        pl.BlockSpec((1, 2 * D_), lambda i: (0, 0)),
    ],
    out_specs=[
        pl.BlockSpec((BLK, 2 * D_), lambda i: (i, 0)),
        pl.BlockSpec((8, 2 * D_), lambda i: (0, 0)),
        pl.BlockSpec((8, 2 * D_), lambda i: (0, 0)),
    ],
    out_shape=[
        jax.ShapeDtypeStruct((NPAD, 2 * D_), jnp.float32),
        jax.ShapeDtypeStruct((8, 2 * D_), jnp.float32),
        jax.ShapeDtypeStruct((8, 2 * D_), jnp.float32),
    ],
)


def _ps2_body(s_ref, ss_ref, sq_ref, g_ref, be_ref, w_ref, b_ref, bt_ref,
              sc_ref, pb_ref, nb_ref):
    i = pl.program_id(0)
    mu = ss_ref[0:1, :] * (1.0 / NVALF)
    var = sq_ref[0:1, :] * (1.0 / NVALF) - mu * mu
    inv = lax.rsqrt(var + 1e-5)
    sn = (s_ref[...] - mu) * (inv * g_ref[...]) + be_ref[...]
    r = jnp.maximum(sn, 0.0)
    v = (jnp.dot(r, w_ref[...], preferred_element_type=jnp.float32)
         + b_ref[...])
    score = jax.nn.sigmoid(v)
    sc_ref[...] = score
    pos = jnp.mean(score, axis=1, keepdims=True)
    bcol = bt_ref[...]
    oneh = (bcol == lax.broadcasted_iota(jnp.int32, (1, B_), 1))
    oneh = oneh.astype(jnp.float32)
    pp = jnp.broadcast_to(
        jnp.sum(pos * oneh, axis=0, keepdims=True), (8, B_))
    nn = jnp.broadcast_to(
        jnp.sum((1.0 - pos) * oneh, axis=0, keepdims=True), (8, B_))

    @pl.when(i == 0)
    def _():
        pb_ref[...] = pp + 1e-8
        nb_ref[...] = nn + 1e-8

    @pl.when(i != 0)
    def _():
        pb_ref[...] += pp
        nb_ref[...] += nn


_ps2 = pl.pallas_call(
    _ps2_body,
    grid=(NB,),
    in_specs=[
        pl.BlockSpec((BLK, 2 * D_), lambda i: (i, 0)),
        pl.BlockSpec((8, 2 * D_), lambda i: (0, 0)),
        pl.BlockSpec((8, 2 * D_), lambda i: (0, 0)),
        pl.BlockSpec((1, 2 * D_), lambda i: (0, 0)),
        pl.BlockSpec((1, 2 * D_), lambda i: (0, 0)),
        pl.BlockSpec((2 * D_, D_), lambda i: (0, 0)),
        pl.BlockSpec((1, D_), lambda i: (0, 0)),
        pl.BlockSpec((BLK, 1), lambda i: (i, 0)),
    ],
    out_specs=[
        pl.BlockSpec((BLK, D_), lambda i: (i, 0)),
        pl.BlockSpec((8, B_), lambda i: (0, 0)),
        pl.BlockSpec((8, B_), lambda i: (0, 0)),
    ],
    out_shape=[
        jax.ShapeDtypeStruct((NPAD, D_), jnp.float32),
        jax.ShapeDtypeStruct((8, B_), jnp.float32),
        jax.ShapeDtypeStruct((8, B_), jnp.float32),
    ],
)


def kernel(x, edge_index, batch, W0, b0, W1, b1, g1, be1, W2, b2, g2, be2,
           eps, Ws1, bs1, gs1, bes1, Ws2, bs2):
    f32 = jnp.float32
    xp = jnp.pad(x, ((0, NPAD - N_), (0, 0)))
    src = edge_index[0].astype(jnp.int32)
    dst = edge_index[1].astype(jnp.int32)
    srcp = jnp.concatenate([src, jnp.zeros((EPAD - E_,), jnp.int32)])
    dstp = jnp.concatenate([dst, jnp.full((EPAD - E_,), TROW, jnp.int32)])
    pk = (srcp | (dstp << 16)).reshape(16, PROW, 2 * CH)
    batchp = jnp.pad(batch.astype(jnp.int32), (0, NPAD - N_),
                     constant_values=B_)[:, None]
    zeros128 = jnp.zeros((NPAD, H), f32)

    h2 = _p0(xp, W0, b0.reshape(1, D_))
    for l in range(L_):
        agg2 = _segsum(h2, pk, zeros128)
        epsl = (1.0 + eps[l]).astype(f32).reshape(1, 1)
        t, ts, tq = _p1(epsl, h2, h2, agg2, agg2, W1[l], b1[l].reshape(1, -1))
        u, us, uq = _p2(t, ts, tq, g1[l].reshape(1, -1), be1[l].reshape(1, -1),
                        W2[l], b2[l].reshape(1, -1))
        h2 = _p3(u, us, uq, g2[l].reshape(1, -1), be2[l].reshape(1, -1))

    s, ss, sq = _ps1(h2, h2, Ws1, bs1.reshape(1, -1))
    scorep, pb, nb = _ps2(s, ss, sq, gs1.reshape(1, -1), bes1.reshape(1, -1),
                          Ws2, bs2.reshape(1, -1), batchp)
    return scorep[:N_], pb[0], nb[0]


# speedup vs baseline: 1.0519x; 1.0157x over previous
import functools

import jax
import jax.numpy as jnp
from jax import lax
from jax.experimental import pallas as pl
from jax.experimental.pallas import tpu as pltpu
from jax.experimental.pallas import tpu_sc as plsc

N_ = 10000
E_ = 160000
D_ = 256
L_ = 3
B_ = 128

NPAD = 10240
BLK = 512
NB = NPAD // BLK
H = 128
CH = 64
NCHUNK = 160
EPT = NCHUNK * CH
EPAD = 16 * EPT
TROW = NPAD - 1
RPT = NPAD // 16
NVALF = float(N_)


NBUF = 4
PROW = NCHUNK // 2


def _sc_body(h_hbm, pk_hbm, zeros_hbm, out_hbm,
             pk, si0, si1, si2, si3, di0, di1, di2, di3,
             r0b, r1b, r2b, r3b, agg_sh,
             sg0, sg1, sg2, sg3, ss0, ss1, ss2, ss3):
    c = lax.axis_index("c")
    s = lax.axis_index("s")
    row0 = s * RPT
    coff = c * NPAD
    pltpu.sync_copy(zeros_hbm.at[pl.ds(row0, RPT)],
                    agg_sh.at[pl.ds(row0, RPT)])
    pltpu.sync_copy(pk_hbm.at[s], pk)
    plsc.subcore_barrier()

    rows = [r0b, r1b, r2b, r3b]
    sis = [si0, si1, si2, si3]
    dis = [di0, di1, di2, di3]
    sg = [sg0, sg1, sg2, sg3]
    ss = [ss0, ss1, ss2, ss3]
    gd = [None] * NBUF
    sd = [None] * NBUF

    def unpack(k):
        b = k % NBUF
        q, half = k // 2, (k % 2) * CH
        for i in range(CH // 16):
            sl = pl.ds(half + i * 16, 16)
            so = pl.ds(i * 16, 16)
            p = pk[q, sl]
            sis[b][0, so] = (p & 0xFFFF) + coff
            dis[b][0, so] = p >> 16

    def start_g(k):
        b = k % NBUF
        gd[b] = pltpu.async_copy(h_hbm.at[sis[b].at[0]], rows[b], sg[b])

    def start_s(k):
        b = k % NBUF
        sd[b] = pltpu.async_copy(rows[b], agg_sh.at[dis[b].at[0]], ss[b],
                                 add=True)

    for j in range(3):
        unpack(j)
        start_g(j)
    for k in range(NCHUNK):
        b = k % NBUF
        gd[b].wait()
        start_s(k)
        kn = k + 3
        if kn < NCHUNK:
            bn = kn % NBUF
            if sd[bn] is not None:
                sd[bn].wait()
                sd[bn] = None
            unpack(kn)
            start_g(kn)
    for b in range(NBUF):
        if sd[b] is not None:
            sd[b].wait()
    plsc.subcore_barrier()
    pltpu.sync_copy(agg_sh.at[pl.ds(row0, RPT)],
                    out_hbm.at[pl.ds(c * NPAD + row0, RPT)])


@functools.cache
def _build_segsum():
    return pl.kernel(
        _sc_body,
        mesh=plsc.VectorSubcoreMesh(core_axis_name="c", subcore_axis_name="s"),
        out_type=jax.ShapeDtypeStruct((2 * NPAD, H), jnp.float32),
        scratch_types=(
            [pltpu.VMEM((PROW, 2 * CH), jnp.int32)]
            + [pltpu.VMEM((1, CH), jnp.int32) for _ in range(2 * NBUF)]
            + [pltpu.VMEM((CH, H), jnp.float32) for _ in range(NBUF)]
            + [pltpu.VMEM_SHARED((NPAD, H), jnp.float32)]
            + [pltpu.SemaphoreType.DMA for _ in range(2 * NBUF)]
        ),
    )


def _segsum(h2, pk, zeros128):
    return _build_segsum()(h2, pk, zeros128)


def _acc_stats(i, val8, ref):
    @pl.when(i == 0)
    def _():
        ref[...] = val8

    @pl.when(i != 0)
    def _():
        ref[...] += val8


def _masked_sums(i, t):
    rid = lax.broadcasted_iota(jnp.int32, (BLK, 1), 0) + i * BLK
    tm = jnp.where(rid < N_, t, 0.0)
    ps = jnp.broadcast_to(jnp.sum(tm, axis=0, keepdims=True), (8, t.shape[1]))
    pq = jnp.broadcast_to(jnp.sum(tm * tm, axis=0, keepdims=True),
                          (8, t.shape[1]))
    return ps, pq


def _p0_body(x_ref, w_ref, b_ref, o_ref):
    o_ref[...] = (jnp.dot(x_ref[...], w_ref[...],
                          preferred_element_type=jnp.float32) + b_ref[...])


_p0 = pl.pallas_call(
    _p0_body,
    grid=(2, NB),
    in_specs=[
        pl.BlockSpec((BLK, D_), lambda h, i: (i, 0)),
        pl.BlockSpec((D_, H), lambda h, i: (0, h)),
        pl.BlockSpec((1, H), lambda h, i: (0, h)),
    ],
    out_specs=pl.BlockSpec((BLK, H), lambda h, i: (h * NB + i, 0)),
    out_shape=jax.ShapeDtypeStruct((2 * NPAD, H), jnp.float32),
)


def _p1_body(eps_ref, ha_ref, hb_ref, aa_ref, ab_ref, w_ref, b_ref,
             t_ref, s_ref, q_ref):
    i = pl.program_id(0)
    e = eps_ref[0, 0]
    za = ha_ref[...] * e + aa_ref[...]
    zb = hb_ref[...] * e + ab_ref[...]
    w = w_ref[...]
    t = (jnp.dot(za, w[:H, :], preferred_element_type=jnp.float32)
         + jnp.dot(zb, w[H:, :], preferred_element_type=jnp.float32)
         + b_ref[...])
    t_ref[...] = t
    ps, pq = _masked_sums(i, t)
    _acc_stats(i, ps, s_ref)
    _acc_stats(i, pq, q_ref)


_p1 = pl.pallas_call(
    _p1_body,
    grid=(NB,),
    in_specs=[
        pl.BlockSpec(memory_space=pltpu.SMEM),
        pl.BlockSpec((BLK, H), lambda i: (i, 0)),
        pl.BlockSpec((BLK, H), lambda i: (NB + i, 0)),
        pl.BlockSpec((BLK, H), lambda i: (i, 0)),
        pl.BlockSpec((BLK, H), lambda i: (NB + i, 0)),
        pl.BlockSpec((D_, 2 * D_), lambda i: (0, 0)),
        pl.BlockSpec((1, 2 * D_), lambda i: (0, 0)),
    ],
    out_specs=[
        pl.BlockSpec((BLK, 2 * D_), lambda i: (i, 0)),
        pl.BlockSpec((8, 2 * D_), lambda i: (0, 0)),
        pl.BlockSpec((8, 2 * D_), lambda i: (0, 0)),
    ],
    out_shape=[
        jax.ShapeDtypeStruct((NPAD, 2 * D_), jnp.float32),
        jax.ShapeDtypeStruct((8, 2 * D_), jnp.float32),
        jax.ShapeDtypeStruct((8, 2 * D_), jnp.float32),
    ],
)


def _p2_body(t_ref, s_ref, q_ref, g_ref, be_ref, w_ref, b_ref,
             u_ref, us_ref, uq_ref):
    i = pl.program_id(0)
    mu = s_ref[0:1, :] * (1.0 / NVALF)
    var = q_ref[0:1, :] * (1.0 / NVALF) - mu * mu
    inv = lax.rsqrt(var + 1e-5)
    tn = (t_ref[...] - mu) * (inv * g_ref[...]) + be_ref[...]
    r = jnp.maximum(tn, 0.0)
    u = (jnp.dot(r, w_ref[...], preferred_element_type=jnp.float32)
         + b_ref[...])
    u_ref[...] = u
    ps, pq = _masked_sums(i, u)
    _acc_stats(i, ps, us_ref)
    _acc_stats(i, pq, uq_ref)


_p2 = pl.pallas_call(
    _p2_body,
    grid=(NB,),
    in_specs=[
        pl.BlockSpec((BLK, 2 * D_), lambda i: (i, 0)),
        pl.BlockSpec((8, 2 * D_), lambda i: (0, 0)),
        pl.BlockSpec((8, 2 * D_), lambda i: (0, 0)),
        pl.BlockSpec((1, 2 * D_), lambda i: (0, 0)),
        pl.BlockSpec((1, 2 * D_), lambda i: (0, 0)),
        pl.BlockSpec((2 * D_, D_), lambda i: (0, 0)),
        pl.BlockSpec((1, D_), lambda i: (0, 0)),
    ],
    out_specs=[
        pl.BlockSpec((BLK, D_), lambda i: (i, 0)),
        pl.BlockSpec((8, D_), lambda i: (0, 0)),
        pl.BlockSpec((8, D_), lambda i: (0, 0)),
    ],
    out_shape=[
        jax.ShapeDtypeStruct((NPAD, D_), jnp.float32),
        jax.ShapeDtypeStruct((8, D_), jnp.float32),
        jax.ShapeDtypeStruct((8, D_), jnp.float32),
    ],
)


def _p3_body(u_ref, s_ref, q_ref, g_ref, be_ref, o_ref):
    mu = s_ref[0:1, :] * (1.0 / NVALF)
    var = q_ref[0:1, :] * (1.0 / NVALF) - mu * mu
    inv = lax.rsqrt(var + 1e-5)
    o_ref[...] = jnp.maximum(
        (u_ref[...] - mu) * (inv * g_ref[...]) + be_ref[...], 0.0)


_p3 = pl.pallas_call(
    _p3_body,
    grid=(2, NB),
    in_specs=[
        pl.BlockSpec((BLK, H), lambda h, i: (i, h)),
        pl.BlockSpec((8, H), lambda h, i: (0, h)),
        pl.BlockSpec((8, H), lambda h, i: (0, h)),
        pl.BlockSpec((1, H), lambda h, i: (0, h)),
        pl.BlockSpec((1, H), lambda h, i: (0, h)),
    ],
    out_specs=pl.BlockSpec((BLK, H), lambda h, i: (h * NB + i, 0)),
    out_shape=jax.ShapeDtypeStruct((2 * NPAD, H), jnp.float32),
)


def _ps1_body(ha_ref, hb_ref, w_ref, b_ref, t_ref, s_ref, q_ref):
    i = pl.program_id(0)
    w = w_ref[...]
    t = (jnp.dot(ha_ref[...], w[:H, :], preferred_element_type=jnp.float32)
         + jnp.dot(hb_ref[...], w[H:, :], preferred_element_type=jnp.float32)
         + b_ref[...])
    t_ref[...] = t
    ps, pq = _masked_sums(i, t)
    _acc_stats(i, ps, s_ref)
    _acc_stats(i, pq, q_ref)


_ps1 = pl.pallas_call(
    _ps1_body,
    grid=(NB,),
    in_specs=[
        pl.BlockSpec((BLK, H), lambda i: (i, 0)),
        pl.BlockSpec((BLK, H), lambda i: (NB + i, 0)),
        pl.BlockSpec((D_, 2 * D_), lambda i: (0, 0)),
        pl.BlockSpec((1, 2 * D_), lambda i: (0, 0)),
    ],
    out_specs=[
        pl.BlockSpec((BLK, 2 * D_), lambda i: (i, 0)),
        pl.BlockSpec((8, 2 * D_), lambda i: (0, 0)),
        pl.BlockSpec((8, 2 * D_), lambda i: (0, 0)),
    ],
    out_shape=[
        jax.ShapeDtypeStruct((NPAD, 2 * D_), jnp.float32),
        jax.ShapeDtypeStruct((8, 2 * D_), jnp.float32),
        jax.ShapeDtypeStruct((8, 2 * D_), jnp.float32),
    ],
)


def _ps2_body(s_ref, ss_ref, sq_ref, g_ref, be_ref, w_ref, b_ref, bt_ref,
              sc_ref, pb_ref, nb_ref):
    i = pl.program_id(0)
    mu = ss_ref[0:1, :] * (1.0 / NVALF)
    var = sq_ref[0:1, :] * (1.0 / NVALF) - mu * mu
    inv = lax.rsqrt(var + 1e-5)
    sn = (s_ref[...] - mu) * (inv * g_ref[...]) + be_ref[...]
    r = jnp.maximum(sn, 0.0)
    v = (jnp.dot(r, w_ref[...], preferred_element_type=jnp.float32)
         + b_ref[...])
    score = jax.nn.sigmoid(v)
    sc_ref[...] = score
    pos = jnp.mean(score, axis=1, keepdims=True)
    bcol = bt_ref[...]
    oneh = (bcol == lax.broadcasted_iota(jnp.int32, (1, B_), 1))
    oneh = oneh.astype(jnp.float32)
    pp = jnp.broadcast_to(
        jnp.sum(pos * oneh, axis=0, keepdims=True), (8, B_))
    nn = jnp.broadcast_to(
        jnp.sum((1.0 - pos) * oneh, axis=0, keepdims=True), (8, B_))

    @pl.when(i == 0)
    def _():
        pb_ref[...] = pp + 1e-8
        nb_ref[...] = nn + 1e-8

    @pl.when(i != 0)
    def _():
        pb_ref[...] += pp
        nb_ref[...] += nn


_ps2 = pl.pallas_call(
    _ps2_body,
    grid=(NB,),
    in_specs=[
        pl.BlockSpec((BLK, 2 * D_), lambda i: (i, 0)),
        pl.BlockSpec((8, 2 * D_), lambda i: (0, 0)),
        pl.BlockSpec((8, 2 * D_), lambda i: (0, 0)),
        pl.BlockSpec((1, 2 * D_), lambda i: (0, 0)),
        pl.BlockSpec((1, 2 * D_), lambda i: (0, 0)),
        pl.BlockSpec((2 * D_, D_), lambda i: (0, 0)),
        pl.BlockSpec((1, D_), lambda i: (0, 0)),
        pl.BlockSpec((BLK, 1), lambda i: (i, 0)),
    ],
    out_specs=[
        pl.BlockSpec((BLK, D_), lambda i: (i, 0)),
        pl.BlockSpec((8, B_), lambda i: (0, 0)),
        pl.BlockSpec((8, B_), lambda i: (0, 0)),
    ],
    out_shape=[
        jax.ShapeDtypeStruct((NPAD, D_), jnp.float32),
        jax.ShapeDtypeStruct((8, B_), jnp.float32),
        jax.ShapeDtypeStruct((8, B_), jnp.float32),
    ],
)


def kernel(x, edge_index, batch, W0, b0, W1, b1, g1, be1, W2, b2, g2, be2,
           eps, Ws1, bs1, gs1, bes1, Ws2, bs2):
    f32 = jnp.float32
    xp = jnp.pad(x, ((0, NPAD - N_), (0, 0)))
    src = edge_index[0].astype(jnp.int32)
    dst = edge_index[1].astype(jnp.int32)
    srcp = jnp.concatenate([src, jnp.zeros((EPAD - E_,), jnp.int32)])
    dstp = jnp.concatenate([dst, jnp.full((EPAD - E_,), TROW, jnp.int32)])
    pk = (srcp | (dstp << 16)).reshape(16, PROW, 2 * CH)
    batchp = jnp.pad(batch.astype(jnp.int32), (0, NPAD - N_),
                     constant_values=B_)[:, None]
    zeros128 = jnp.zeros((NPAD, H), f32)

    h2 = _p0(xp, W0, b0.reshape(1, D_))
    for l in range(L_):
        agg2 = _segsum(h2, pk, zeros128)
        epsl = (1.0 + eps[l]).astype(f32).reshape(1, 1)
        t, ts, tq = _p1(epsl, h2, h2, agg2, agg2, W1[l], b1[l].reshape(1, -1))
        u, us, uq = _p2(t, ts, tq, g1[l].reshape(1, -1), be1[l].reshape(1, -1),
                        W2[l], b2[l].reshape(1, -1))
        h2 = _p3(u, us, uq, g2[l].reshape(1, -1), be2[l].reshape(1, -1))

    s, ss, sq = _ps1(h2, h2, Ws1, bs1.reshape(1, -1))
    scorep, pb, nb = _ps2(s, ss, sq, gs1.reshape(1, -1), bes1.reshape(1, -1),
                          Ws2, bs2.reshape(1, -1), batchp)
    return scorep[:N_], pb[0], nb[0]
